# baseline (speedup 1.0000x reference)
import jax
import jax.numpy as jnp
from jax.experimental import pallas as pl
from jax.experimental.pallas import tpu as pltpu

B_TOTAL = 4096
BT = 16
N_TILES = B_TOTAL // BT

_T = ({1: 1, 0: 3}, {2: 0, 1: 2})
_R = {(0, 0): ((0, 1, 1), (1, 0, 3)),
      (1, 0): ((1, 1, 1), (0, 1, 3)),
      (0, 1): ((1, 1, 0), (0, 1, 2)),
      (1, 1): ((0, 2, 0), (1, 1, 2))}
_W4META = [(2 * r1 + s1, a, b)
           for r1 in range(2) for s1 in range(2)
           for a in ((1, 2) if r1 == 0 else (0, 1))
           for b in ((1, 2) if s1 == 0 else (0, 1))]


def _fwd_kernel(x_ref, w1_ref, b1_ref, w2_ref, b2_ref, embt_ref,
                w3_ref, b3_ref, w4_ref, b4_ref,
                xr_ref, loss_ref, perp_ref, enc_ref,
                loss_acc, cnt_acc, s_z1, s_acc2, s_acc3, s_acc4):
    i = pl.program_id(0)

    @pl.when(i == 0)
    def _init():
        loss_acc[...] = jnp.zeros((1, 1), jnp.float32)
        cnt_acc[...] = jnp.zeros((1, 10), jnp.float32)

    x = x_ref[...]

    xp = jnp.pad(x, ((0, 0), (1, 1), (1, 1)))
    pr = xp.reshape(BT, 15, 2, 30)
    planes1 = []
    for r in range(2):
        row = pr[:, :, r, :].reshape(BT, 15, 15, 2)
        planes1.append([row[:, :, :, 0], row[:, :, :, 1]])
    s_z1[...] = jnp.broadcast_to(b1_ref[0][None, None, None, :],
                                 (BT, 14, 14, 32))
    for kh in range(4):
        for kw in range(4):
            tap = planes1[kh % 2][kw % 2][:, kh // 2:kh // 2 + 14,
                                          kw // 2:kw // 2 + 14]
            s_z1[...] += tap[..., None] * w1_ref[kh * 4 + kw][None, None,
                                                              None, :]
    z1 = jnp.maximum(s_z1[...], 0.0)

    z1p = jnp.pad(z1, ((0, 0), (1, 1), (1, 1), (0, 0)))
    pr2 = z1p.reshape(BT, 8, 2, 16, 32)
    planes2 = []
    for r in range(2):
        row = pr2[:, :, r, :, :].reshape(BT, 8, 8, 2, 32)
        planes2.append([row[:, :, :, 0, :], row[:, :, :, 1, :]])
    s_acc2[...] = jnp.broadcast_to(b2_ref[0][None, :], (BT * 49, 64))
    for kh in range(4):
        for kw in range(4):
            tap = planes2[kh % 2][kw % 2][:, kh // 2:kh // 2 + 7,
                                          kw // 2:kw // 2 + 7, :]
            s_acc2[...] += jnp.dot(tap.reshape(BT * 49, 32),
                                   w2_ref[kh * 4 + kw],
                                   preferred_element_type=jnp.float32)
    z2 = jnp.maximum(s_acc2[...], 0.0)

    z2v3 = z2.reshape(BT, 49, 64)
    embr = embt_ref[...]
    dots = jnp.zeros((BT, 10), jnp.float32)
    for p in range(49):
        dots = dots + jnp.dot(z2v3[:, p, :].reshape(BT, 64),
                              embr[p * 64:(p + 1) * 64, :],
                              preferred_element_type=jnp.float32)
    znorm = jnp.sum(jnp.sum(z2v3 * z2v3, axis=2), axis=1, keepdims=True)
    enorm = jnp.sum(embr * embr, axis=0)[None, :]
    d = znorm + enorm - 2.0 * dots
    idx = jnp.argmin(d, axis=1)
    enc = (jax.lax.broadcasted_iota(jnp.int32, (BT, 10), 1)
           == idx[:, None]).astype(jnp.float32)
    enc_ref[...] = enc
    loss_acc[...] += jnp.sum(jnp.min(d, axis=1)).reshape(1, 1)
    cnt_acc[...] += jnp.sum(enc, axis=0, keepdims=True)

    z2v4 = z2.reshape(BT, 7, 7, 64)
    z2p = jnp.pad(z2v4, ((0, 0), (1, 1), (1, 1), (0, 0)))
    s_acc3[...] = jnp.broadcast_to(b3_ref[0][None, :], (BT * 49, 128))
    for o in range(9):
        a, b = o // 3, o % 3
        sl = z2p[:, a:a + 7, b:b + 7, :].reshape(BT * 49, 64)
        s_acc3[...] += jnp.dot(sl, w3_ref[o],
                               preferred_element_type=jnp.float32)
    hp4 = jnp.maximum(s_acc3[...], 0.0).reshape(BT, 7, 7, 128)
    hpp = jnp.pad(hp4, ((0, 0), (1, 1), (1, 1), (0, 0)))

    s_acc4[...] = jnp.zeros((BT * 49, 16), jnp.float32)
    for k, (pi, a, b) in enumerate(_W4META):
        sl = hpp[:, a:a + 7, b:b + 7,
                 32 * pi:32 * pi + 32].reshape(BT * 49, 32)
        s_acc4[...] += jnp.dot(sl, w4_ref[k],
                               preferred_element_type=jnp.float32)
    xr_ref[...] = jax.nn.sigmoid(s_acc4[...] + b4_ref[...]) \
        .reshape(BT, 7, 7, 16)

    @pl.when(i == N_TILES - 1)
    def _fin():
        loss_ref[...] = loss_acc[...] * (1.25 / (B_TOTAL * 3136.0))
        p = cnt_acc[...] / B_TOTAL
        perp_ref[...] = jnp.exp(-jnp.sum(p * jnp.log(p + 1e-10))).reshape(1, 1)


def kernel(x, conv1_w, conv1_b, conv2_w, conv2_b, emb,
           dec1_w, dec1_b, dec2_w, dec2_b):
    x3 = x.reshape(B_TOTAL, 28, 28)
    w1 = conv1_w[:, 0].transpose(1, 2, 0).reshape(16, 32)
    w2 = conv2_w.transpose(2, 3, 1, 0).reshape(16, 32, 64)
    embt = emb.T
    w3t = dec1_w.transpose(2, 3, 0, 1)
    w4t = dec2_w[:, 0].transpose(1, 2, 0)

    w3e = []
    for a in range(3):
        for b in range(3):
            m = jnp.zeros((64, 128), jnp.float32)
            for r in range(2):
                kh = _T[r].get(a)
                if kh is None:
                    continue
                for s in range(2):
                    kw = _T[s].get(b)
                    if kw is None:
                        continue
                    c = 32 * (2 * r + s)
                    m = m.at[:, c:c + 32].set(w3t[kh, kw])
            w3e.append(m)
    w3e = jnp.stack(w3e)
    b3t = jnp.tile(dec1_b, 4)[None, :]

    w4e = []
    for pi, a, b in _W4META:
        r1, s1 = pi // 2, pi % 2
        m = jnp.zeros((32, 16), jnp.float32)
        for (rho, r2), rents in _R.items():
            for (rr1, ra, kh) in rents:
                if (rr1, ra) != (r1, a):
                    continue
                for (sig, s2), cents in _R.items():
                    for (ss1, sb, kw) in cents:
                        if (ss1, sb) != (s1, b):
                            continue
                        j = ((rho * 2 + r2) * 2 + sig) * 2 + s2
                        m = m.at[:, j].set(w4t[kh, kw])
        w4e.append(m)
    w4e = jnp.stack(w4e)

    grid = (N_TILES,)
    xr, loss, perp, enc = pl.pallas_call(
        _fwd_kernel,
        grid=grid,
        in_specs=[
            pl.BlockSpec((BT, 28, 28), lambda i: (i, 0, 0)),
            pl.BlockSpec((16, 32), lambda i: (0, 0)),
            pl.BlockSpec((1, 32), lambda i: (0, 0)),
            pl.BlockSpec((16, 32, 64), lambda i: (0, 0, 0)),
            pl.BlockSpec((1, 64), lambda i: (0, 0)),
            pl.BlockSpec((3136, 10), lambda i: (0, 0)),
            pl.BlockSpec((9, 64, 128), lambda i: (0, 0, 0)),
            pl.BlockSpec((1, 128), lambda i: (0, 0)),
            pl.BlockSpec((16, 32, 16), lambda i: (0, 0, 0)),
            pl.BlockSpec((1, 1), lambda i: (0, 0)),
        ],
        out_specs=[
            pl.BlockSpec((BT, 7, 7, 16), lambda i: (i, 0, 0, 0)),
            pl.BlockSpec((1, 1), lambda i: (0, 0)),
            pl.BlockSpec((1, 1), lambda i: (0, 0)),
            pl.BlockSpec((BT, 10), lambda i: (i, 0)),
        ],
        out_shape=[
            jax.ShapeDtypeStruct((B_TOTAL, 7, 7, 16), jnp.float32),
            jax.ShapeDtypeStruct((1, 1), jnp.float32),
            jax.ShapeDtypeStruct((1, 1), jnp.float32),
            jax.ShapeDtypeStruct((B_TOTAL, 10), jnp.float32),
        ],
        scratch_shapes=[
            pltpu.VMEM((1, 1), jnp.float32),
            pltpu.VMEM((1, 10), jnp.float32),
            pltpu.VMEM((BT, 14, 14, 32), jnp.float32),
            pltpu.VMEM((BT * 49, 64), jnp.float32),
            pltpu.VMEM((BT * 49, 128), jnp.float32),
            pltpu.VMEM((BT * 49, 16), jnp.float32),
        ],
        compiler_params=pltpu.CompilerParams(
            dimension_semantics=("arbitrary",)),
    )(x3, w1, conv1_b[None, :], w2, conv2_b[None, :], embt,
      w3e, b3t, w4e, dec2_b[None, :])

    xrec = xr.reshape(B_TOTAL, 7, 7, 2, 2, 2, 2) \
             .transpose(0, 1, 3, 4, 2, 5, 6).reshape(B_TOTAL, 1, 28, 28)
    return (xrec, loss[0, 0], perp[0, 0], enc)


# speedup vs baseline: 9.6983x; 9.6983x over previous
import numpy as np
import jax
import jax.numpy as jnp
from jax.experimental import pallas as pl
from jax.experimental.pallas import tpu as pltpu

B_TOTAL = 4096
BT = 64
N_TILES = B_TOTAL // BT

_CT_TAPS = (((1, 1), (3, 0)), ((0, 2), (2, 1)))


def _sel_conv(n_in, n_out, kw):
    s = np.zeros((n_in, n_out), np.float32)
    for ow in range(n_out):
        j = 2 * ow + kw - 1
        if 0 <= j < n_in:
            s[j, ow] = 1.0
    return s


def _sel_convt(n_in, n_out, kw):
    s = np.zeros((n_in, n_out), np.float32)
    for iw in range(n_in):
        k = np.arange(n_out) - 2 * iw + 1
        for ow in range(n_out):
            if k[ow] == kw:
                s[iw, ow] = 1.0
    return s


def _fwd_kernel(x_ref, m1_ref, b1_ref, m2_ref, b2_ref, embt_ref,
                m3_ref, b3_ref, m4_ref, b4_ref,
                xr_ref, loss_ref, perp_ref, enc_ref,
                loss_acc, cnt_acc):
    i = pl.program_id(0)

    @pl.when(i == 0)
    def _init():
        loss_acc[...] = jnp.zeros((1, 1), jnp.float32)
        cnt_acc[...] = jnp.zeros((1, 10), jnp.float32)

    x = x_ref[...]

    xph = jnp.pad(x, ((0, 0), (1, 1), (0, 0)))
    pr = xph.reshape(BT, 15, 2, 28)
    p1 = (pr[:, :, 0, :], pr[:, :, 1, :])
    y1 = jnp.broadcast_to(b1_ref[...], (BT * 14, 448))
    for kh in range(4):
        v = p1[kh % 2][:, kh // 2:kh // 2 + 14, :].reshape(BT * 14, 28)
        y1 = y1 + jnp.dot(v, m1_ref[kh], preferred_element_type=jnp.float32)
    y1 = jnp.maximum(y1, 0.0).reshape(BT, 14, 448)

    y1p = jnp.pad(y1, ((0, 0), (1, 1), (0, 0)))
    pr2 = y1p.reshape(BT, 8, 2, 448)
    p2 = (pr2[:, :, 0, :], pr2[:, :, 1, :])
    z2 = jnp.broadcast_to(b2_ref[...], (BT * 7, 448))
    for kh in range(4):
        v = p2[kh % 2][:, kh // 2:kh // 2 + 7, :].reshape(BT * 7, 448)
        z2 = z2 + jnp.dot(v, m2_ref[kh], preferred_element_type=jnp.float32)
    z2 = jnp.maximum(z2, 0.0).reshape(BT, 7, 448)

    embt = embt_ref[...]
    dots = jnp.zeros((BT, 10), jnp.float32)
    for oh in range(7):
        dots = dots + jnp.dot(z2[:, oh, :].reshape(BT, 448),
                              embt[oh * 448:(oh + 1) * 448, :],
                              preferred_element_type=jnp.float32)
    znorm = jnp.sum(jnp.sum(z2 * z2, axis=2), axis=1, keepdims=True)
    enorm = jnp.sum(embt * embt, axis=0)[None, :]
    d = znorm + enorm - 2.0 * dots
    idx = jnp.argmin(d, axis=1)
    enc = (jax.lax.broadcasted_iota(jnp.int32, (BT, 10), 1)
           == idx[:, None]).astype(jnp.float32)
    enc_ref[...] = enc
    loss_acc[...] += jnp.sum(jnp.min(d, axis=1)).reshape(1, 1)
    cnt_acc[...] += jnp.sum(enc, axis=0, keepdims=True)

    z2p = jnp.pad(z2, ((0, 0), (1, 1), (0, 0)))
    hplanes = []
    for r in range(2):
        a = jnp.broadcast_to(b3_ref[...], (BT * 7, 448))
        for kh, ro in _CT_TAPS[r]:
            v = z2p[:, ro:ro + 7, :].reshape(BT * 7, 448)
            a = a + jnp.dot(v, m3_ref[kh],
                            preferred_element_type=jnp.float32)
        hplanes.append(jnp.maximum(a, 0.0).reshape(BT, 7, 448))
    h = jnp.stack(hplanes, axis=2).reshape(BT, 14, 448)

    hp = jnp.pad(h, ((0, 0), (1, 1), (0, 0)))
    for r2 in range(2):
        a = jnp.zeros((BT * 14, 28), jnp.float32)
        for kh, ro in _CT_TAPS[r2]:
            v = hp[:, ro:ro + 14, :].reshape(BT * 14, 448)
            a = a + jnp.dot(v, m4_ref[kh],
                            preferred_element_type=jnp.float32)
        xr_ref[:, r2, :, :] = jax.nn.sigmoid(a + b4_ref[...]) \
            .reshape(BT, 14, 28)

    @pl.when(i == N_TILES - 1)
    def _fin():
        loss_ref[...] = loss_acc[...] * (1.25 / (B_TOTAL * 3136.0))
        p = cnt_acc[...] / B_TOTAL
        perp_ref[...] = jnp.exp(-jnp.sum(p * jnp.log(p + 1e-10))).reshape(1, 1)


def kernel(x, conv1_w, conv1_b, conv2_w, conv2_b, emb,
           dec1_w, dec1_b, dec2_w, dec2_b):
    x3 = x.reshape(B_TOTAL, 28, 28)
    embt = emb.T

    m1 = jnp.stack([
        sum(jnp.asarray(_sel_conv(28, 14, kw))[:, :, None]
            * conv1_w[:, 0, kh, kw][None, None, :] for kw in range(4))
        .reshape(28, 448) for kh in range(4)])
    m2 = jnp.stack([
        sum(jnp.asarray(_sel_conv(14, 7, kw))[:, None, :, None]
            * conv2_w[:, :, kh, kw].T[None, :, None, :] for kw in range(4))
        .reshape(448, 448) for kh in range(4)])
    m3 = jnp.stack([
        sum(jnp.asarray(_sel_convt(7, 14, kw))[:, None, :, None]
            * dec1_w[:, :, kh, kw][None, :, None, :] for kw in range(4))
        .reshape(448, 448) for kh in range(4)])
    m4 = jnp.stack([
        sum(jnp.asarray(_sel_convt(14, 28, kw))[:, None, :]
            * dec2_w[:, 0, kh, kw][None, :, None] for kw in range(4))
        .reshape(448, 28) for kh in range(4)])
    b1t = jnp.tile(conv1_b, 14)[None, :]
    b2t = jnp.tile(conv2_b, 7)[None, :]
    b3t = jnp.tile(dec1_b, 14)[None, :]

    grid = (N_TILES,)
    xr, loss, perp, enc = pl.pallas_call(
        _fwd_kernel,
        grid=grid,
        in_specs=[
            pl.BlockSpec((BT, 28, 28), lambda i: (i, 0, 0)),
            pl.BlockSpec((4, 28, 448), lambda i: (0, 0, 0)),
            pl.BlockSpec((1, 448), lambda i: (0, 0)),
            pl.BlockSpec((4, 448, 448), lambda i: (0, 0, 0)),
            pl.BlockSpec((1, 448), lambda i: (0, 0)),
            pl.BlockSpec((3136, 10), lambda i: (0, 0)),
            pl.BlockSpec((4, 448, 448), lambda i: (0, 0, 0)),
            pl.BlockSpec((1, 448), lambda i: (0, 0)),
            pl.BlockSpec((4, 448, 28), lambda i: (0, 0, 0)),
            pl.BlockSpec((1, 1), lambda i: (0, 0)),
        ],
        out_specs=[
            pl.BlockSpec((BT, 2, 14, 28), lambda i: (i, 0, 0, 0)),
            pl.BlockSpec((1, 1), lambda i: (0, 0)),
            pl.BlockSpec((1, 1), lambda i: (0, 0)),
            pl.BlockSpec((BT, 10), lambda i: (i, 0)),
        ],
        out_shape=[
            jax.ShapeDtypeStruct((B_TOTAL, 2, 14, 28), jnp.float32),
            jax.ShapeDtypeStruct((1, 1), jnp.float32),
            jax.ShapeDtypeStruct((1, 1), jnp.float32),
            jax.ShapeDtypeStruct((B_TOTAL, 10), jnp.float32),
        ],
        scratch_shapes=[
            pltpu.VMEM((1, 1), jnp.float32),
            pltpu.VMEM((1, 10), jnp.float32),
        ],
        compiler_params=pltpu.CompilerParams(
            dimension_semantics=("arbitrary",)),
    )(x3, m1, b1t, m2, b2t, embt, m3, b3t, m4, dec2_b[None, :])

    xrec = xr.transpose(0, 2, 1, 3).reshape(B_TOTAL, 1, 28, 28)
    return (xrec, loss[0, 0], perp[0, 0], enc)
